# baseline (speedup 1.0000x reference)
import jax
import jax.numpy as jnp
from jax.experimental import pallas as pl

DEPTH = 1000
BATCH = 4096
HIST = 20
BBLK = 32


def _onehot_body(idx_ref, out_ref):
    d = jax.lax.broadcasted_iota(jnp.int32, (BBLK, DEPTH), 1)
    for h in range(HIST):
        col = idx_ref[:, h : h + 1]
        out_ref[:, h, :] = jnp.where(d == col, 1.0, 0.0).astype(jnp.float32)


def kernel(input, emb_weight):
    del emb_weight
    return pl.pallas_call(
        _onehot_body,
        grid=(BATCH // BBLK,),
        in_specs=[pl.BlockSpec((BBLK, HIST), lambda i: (i, 0))],
        out_specs=pl.BlockSpec((BBLK, HIST, DEPTH), lambda i: (i, 0, 0)),
        out_shape=jax.ShapeDtypeStruct((BATCH, HIST, DEPTH), jnp.float32),
    )(input)


# speedup vs baseline: 1.0686x; 1.0686x over previous
import jax
import jax.numpy as jnp
from jax.experimental import pallas as pl

DEPTH = 1000
BATCH = 4096
HIST = 20
BBLK = 256


def _onehot_body(idx_ref, out_ref):
    d = jax.lax.broadcasted_iota(jnp.int32, (BBLK, DEPTH), 1)
    for h in range(HIST):
        col = idx_ref[:, h : h + 1]
        out_ref[:, h, :] = jnp.where(d == col, 1.0, 0.0).astype(jnp.float32)


def kernel(input, emb_weight):
    del emb_weight
    return pl.pallas_call(
        _onehot_body,
        grid=(BATCH // BBLK,),
        in_specs=[pl.BlockSpec((BBLK, HIST), lambda i: (i, 0))],
        out_specs=pl.BlockSpec((BBLK, HIST, DEPTH), lambda i: (i, 0, 0)),
        out_shape=jax.ShapeDtypeStruct((BATCH, HIST, DEPTH), jnp.float32),
    )(input)
